# baseline (speedup 1.0000x reference)
import functools

import jax
import jax.numpy as jnp
from jax import lax
from jax.experimental import pallas as pl
from jax.experimental.pallas import tpu as pltpu
from jax.experimental.pallas import tpu_sc as plsc

_CHUNK = 128


@functools.lru_cache(maxsize=None)
def _make_gather(num_rows, dim, batch):
    info = plsc.get_sparse_core_info()
    nw = info.num_cores * info.num_subcores
    b_per_w = batch // nw
    n_chunks = b_per_w // _CHUNK
    mesh = plsc.VectorSubcoreMesh(core_axis_name="c", subcore_axis_name="s")

    @functools.partial(
        pl.kernel,
        mesh=mesh,
        compiler_params=pltpu.CompilerParams(use_tc_tiling_on_sc=False),
        out_type=jax.ShapeDtypeStruct((dim, batch), jnp.float32),
        scratch_types=[
            pltpu.VMEM((n_chunks, _CHUNK), jnp.int32),
            pltpu.VMEM((dim, b_per_w), jnp.float32),
            pltpu.SemaphoreType.DMA,
        ],
    )
    def k(idx_hbm, tablet_hbm, outt_hbm, idx_v, rows_v, sem):
        wid = lax.axis_index("s") * info.num_cores + lax.axis_index("c")
        base = wid * b_per_w
        pltpu.sync_copy(idx_hbm.at[pl.ds(wid * n_chunks, n_chunks)], idx_v)
        for j in range(n_chunks):
            copies = []
            for d in range(dim):
                copies.append(
                    pltpu.async_copy(
                        tablet_hbm.at[d].at[idx_v.at[j]],
                        rows_v.at[d, pl.ds(j * _CHUNK, _CHUNK)],
                        sem,
                    )
                )
            for c in copies:
                c.wait()
        pltpu.sync_copy(rows_v, outt_hbm.at[:, pl.ds(base, b_per_w)])

    return k


def kernel(x, table):
    batch = x.shape[0]
    num_rows, dim = table.shape
    xr = jnp.reshape(x, (batch // _CHUNK, _CHUNK))
    outt = _make_gather(num_rows, dim, batch)(xr, table.T)
    return outt.T


# speedup vs baseline: 17.4618x; 17.4618x over previous
import functools

import jax
import jax.numpy as jnp
from jax import lax
from jax.experimental import pallas as pl
from jax.experimental.pallas import tpu as pltpu
from jax.experimental.pallas import tpu_sc as plsc

_G = 16
_LANES = 128


@functools.lru_cache(maxsize=None)
def _make_gather(num_rows, dim, batch):
    info = plsc.get_sparse_core_info()
    nw = info.num_cores * info.num_subcores
    b_per_w = batch // nw
    n_groups = b_per_w // _G
    mesh = plsc.VectorSubcoreMesh(core_axis_name="c", subcore_axis_name="s")

    @functools.partial(
        pl.kernel,
        mesh=mesh,
        compiler_params=pltpu.CompilerParams(needs_layout_passes=False),
        out_type=jax.ShapeDtypeStruct((batch, _LANES), jnp.float32),
        scratch_types=[
            pltpu.VMEM((b_per_w,), jnp.int32),
            pltpu.VMEM((b_per_w,), jnp.int32),
            pltpu.VMEM((b_per_w,), jnp.int32),
            pltpu.VMEM((_G, dim, _LANES), jnp.float32),
            pltpu.VMEM((2, _G, _LANES), jnp.float32),
            pltpu.SemaphoreType.DMA,
            pltpu.SemaphoreType.DMA,
        ],
    )
    def k(idx_hbm, tt_hbm, out_hbm, idx_v, blks_v, lanes_v, banks, stage,
          semg, semo):
        wid = lax.axis_index("s") * info.num_cores + lax.axis_index("c")
        base = wid * b_per_w
        pltpu.sync_copy(idx_hbm.at[pl.ds(base, b_per_w)], idx_v)
        d16 = lax.iota(jnp.int32, _G)
        zeros = jnp.zeros((_G,), jnp.int32)
        for g in range(n_groups):
            vec = idx_v[pl.ds(g * _G, _G)]
            blks_v[pl.ds(g * _G, _G)] = lax.shift_right_logical(vec, 7)
            lanes_v[pl.ds(g * _G, _G)] = lax.bitwise_and(vec, _LANES - 1)

        def body(p, _):
            i0 = p * _G
            bvec = blks_v[pl.ds(i0, _G)]
            lvec = lanes_v[pl.ds(i0, _G)]
            copies = []
            for j in range(_G):
                col0 = pl.multiple_of(bvec[j] * _LANES, _LANES)
                copies.append(
                    pltpu.async_copy(
                        tt_hbm.at[:, pl.ds(col0, _LANES)], banks.at[j], semg
                    )
                )
            for c in copies:
                c.wait()
            sb = stage.at[lax.rem(p, 2)]
            for j in range(_G):
                lane = zeros + lvec[j]
                for m in range(dim // _G):
                    col = plsc.load_gather(banks.at[j], [d16 + _G * m, lane])
                    sb[j, pl.ds(m * _G, _G)] = col
            @pl.when(p >= 2)
            def _():
                pltpu.make_async_copy(
                    sb, out_hbm.at[pl.ds(base, _G), :], semo
                ).wait()

            pltpu.async_copy(sb, out_hbm.at[pl.ds(base + i0, _G), :], semo)
            return 0

        lax.fori_loop(0, n_groups, body, 0)
        for _ in range(2):
            pltpu.make_async_copy(
                stage.at[0], out_hbm.at[pl.ds(base, _G), :], semo
            ).wait()

    return k


def kernel(x, table):
    batch = x.shape[0]
    num_rows, dim = table.shape
    out = _make_gather(num_rows, dim, batch)(x, table.T)
    return out[:, :dim]


# speedup vs baseline: 19.5546x; 1.1199x over previous
import functools

import jax
import jax.numpy as jnp
from jax import lax
from jax.experimental import pallas as pl
from jax.experimental.pallas import tpu as pltpu
from jax.experimental.pallas import tpu_sc as plsc

_G = 16
_LANES = 128
_DH = 16


@functools.lru_cache(maxsize=None)
def _make_gather(num_rows, dim, batch):
    info = plsc.get_sparse_core_info()
    nw = info.num_cores * info.num_subcores
    b_per_w = batch // nw
    n_groups = b_per_w // _G
    mesh = plsc.VectorSubcoreMesh(core_axis_name="c", subcore_axis_name="s")

    @functools.partial(
        pl.kernel,
        mesh=mesh,
        compiler_params=pltpu.CompilerParams(needs_layout_passes=False),
        out_type=jax.ShapeDtypeStruct((dim, batch), jnp.float32),
        scratch_types=[
            pltpu.VMEM((b_per_w,), jnp.int32),
            pltpu.VMEM((b_per_w,), jnp.int32),
            pltpu.VMEM((b_per_w,), jnp.int32),
            pltpu.VMEM((_G, _DH, _LANES), jnp.float32),
            pltpu.VMEM((_G, _DH, _LANES), jnp.float32),
            pltpu.VMEM((dim, b_per_w), jnp.float32),
            pltpu.SemaphoreType.DMA,
            pltpu.SemaphoreType.DMA,
        ],
    )
    def k(idx_hbm, tt_hbm, out_hbm, idx_v, blks_v, lanes_v, bank_a, bank_b,
          rows_v, sem_a, sem_b):
        wid = lax.axis_index("s") * info.num_cores + lax.axis_index("c")
        base = wid * b_per_w
        pltpu.sync_copy(idx_hbm.at[pl.ds(base, b_per_w)], idx_v)
        d16 = lax.iota(jnp.int32, _G)
        for g in range(n_groups):
            vec = idx_v[pl.ds(g * _G, _G)]
            blks_v[pl.ds(g * _G, _G)] = lax.shift_right_logical(vec, 7)
            lanes_v[pl.ds(g * _G, _G)] = lax.bitwise_and(vec, _LANES - 1)

        def fire(g, dh_static, bank, sem):
            bvec = blks_v[pl.ds(g * _G, _G)]
            copies = []
            for j in range(_G):
                col0 = pl.multiple_of(bvec[j] * _LANES, _LANES)
                copies.append(
                    pltpu.async_copy(
                        tt_hbm.at[pl.ds(dh_static * _DH, _DH),
                                  pl.ds(col0, _LANES)],
                        bank.at[j],
                        sem,
                    )
                )
            return copies

        def drain(bank, sem):
            for _ in range(_G):
                pltpu.make_async_copy(
                    tt_hbm.at[pl.ds(0, _DH), pl.ds(0, _LANES)],
                    bank.at[0],
                    sem,
                ).wait()

        def extract(g, dh_static, bank):
            lvec = lanes_v[pl.ds(g * _G, _G)]
            for dl in range(_DH):
                dvec = jnp.full((_G,), dl, jnp.int32)
                vals = plsc.load_gather(bank, [d16, dvec, lvec])
                rows_v[dh_static * _DH + dl, pl.ds(g * _G, _G)] = vals

        fire(0, 0, bank_a, sem_a)

        def body(v, _):
            fire(v, 1, bank_b, sem_b)
            drain(bank_a, sem_a)
            extract(v, 0, bank_a)

            @pl.when(v < n_groups - 1)
            def _():
                fire(v + 1, 0, bank_a, sem_a)

            drain(bank_b, sem_b)
            extract(v, 1, bank_b)
            return 0

        lax.fori_loop(0, n_groups, body, 0)
        pltpu.sync_copy(rows_v, out_hbm.at[:, pl.ds(base, b_per_w)])

    return k


def kernel(x, table):
    batch = x.shape[0]
    num_rows, dim = table.shape
    outt = _make_gather(num_rows, dim, batch)(x, table.T)
    return outt.T


# speedup vs baseline: 21.0937x; 1.0787x over previous
import functools

import jax
import jax.numpy as jnp
from jax import lax
from jax.experimental import pallas as pl
from jax.experimental.pallas import tpu as pltpu
from jax.experimental.pallas import tpu_sc as plsc

_G = 16
_LANES = 128
_DQ = 8
_NB = 4


@functools.lru_cache(maxsize=None)
def _make_gather(num_rows, dim, batch):
    info = plsc.get_sparse_core_info()
    nw = info.num_cores * info.num_subcores
    b_per_w = batch // nw
    n_groups = b_per_w // _G
    mesh = plsc.VectorSubcoreMesh(core_axis_name="c", subcore_axis_name="s")

    @functools.partial(
        pl.kernel,
        mesh=mesh,
        compiler_params=pltpu.CompilerParams(needs_layout_passes=False),
        out_type=jax.ShapeDtypeStruct((dim, batch), jnp.float32),
        scratch_types=[
            pltpu.VMEM((b_per_w,), jnp.int32),
            pltpu.VMEM((b_per_w,), jnp.int32),
            pltpu.VMEM((b_per_w,), jnp.int32),
            [pltpu.VMEM((_G, _DQ, _LANES), jnp.float32) for _ in range(_NB)],
            pltpu.VMEM((dim, b_per_w), jnp.float32),
            [pltpu.SemaphoreType.DMA for _ in range(_NB)],
        ],
    )
    def k(idx_hbm, tt_hbm, out_hbm, idx_v, blks_v, lanes_v, banks, rows_v,
          sems):
        wid = lax.axis_index("s") * info.num_cores + lax.axis_index("c")
        base = wid * b_per_w
        pltpu.sync_copy(idx_hbm.at[pl.ds(base, b_per_w)], idx_v)
        d16 = lax.iota(jnp.int32, _G)
        for g in range(n_groups):
            vec = idx_v[pl.ds(g * _G, _G)]
            blks_v[pl.ds(g * _G, _G)] = lax.shift_right_logical(vec, 7)
            lanes_v[pl.ds(g * _G, _G)] = lax.bitwise_and(vec, _LANES - 1)

        def fire(g, dq):
            bvec = blks_v[pl.ds(g * _G, _G)]
            for j in range(_G):
                col0 = pl.multiple_of(bvec[j] * _LANES, _LANES)
                pltpu.async_copy(
                    tt_hbm.at[pl.ds(dq * _DQ, _DQ), pl.ds(col0, _LANES)],
                    banks[dq].at[j],
                    sems[dq],
                )

        def drain(dq):
            for _ in range(_G):
                pltpu.make_async_copy(
                    tt_hbm.at[pl.ds(0, _DQ), pl.ds(0, _LANES)],
                    banks[dq].at[0],
                    sems[dq],
                ).wait()

        def extract(g, dq):
            lvec = lanes_v[pl.ds(g * _G, _G)]
            for dl in range(_DQ):
                dvec = jnp.full((_G,), dl, jnp.int32)
                vals = plsc.load_gather(banks[dq], [d16, dvec, lvec])
                rows_v[dq * _DQ + dl, pl.ds(g * _G, _G)] = vals

        for dq in range(_NB):
            fire(0, dq)

        def body(v, _):
            for dq in range(_NB):
                drain(dq)
                extract(v, dq)

                @pl.when(v < n_groups - 1)
                def _():
                    fire(v + 1, dq)

            return 0

        lax.fori_loop(0, n_groups, body, 0)
        pltpu.sync_copy(rows_v, out_hbm.at[:, pl.ds(base, b_per_w)])

    return k


def kernel(x, table):
    batch = x.shape[0]
    num_rows, dim = table.shape
    outt = _make_gather(num_rows, dim, batch)(x, table.T)
    return outt.T


# speedup vs baseline: 21.1203x; 1.0013x over previous
import functools

import jax
import jax.numpy as jnp
from jax import lax
from jax.experimental import pallas as pl
from jax.experimental.pallas import tpu as pltpu
from jax.experimental.pallas import tpu_sc as plsc

_G = 16
_LANES = 128
_DQ = 8
_NB = 6
_UPB = 12


@functools.lru_cache(maxsize=None)
def _make_gather(num_rows, dim, batch):
    info = plsc.get_sparse_core_info()
    nw = info.num_cores * info.num_subcores
    b_per_w = batch // nw
    n_groups = b_per_w // _G
    nq = dim // _DQ
    n_units = n_groups * nq
    n_bodies = (n_units - _NB - (_UPB - _NB)) // _UPB
    mesh = plsc.VectorSubcoreMesh(core_axis_name="c", subcore_axis_name="s")

    @functools.partial(
        pl.kernel,
        mesh=mesh,
        compiler_params=pltpu.CompilerParams(needs_layout_passes=False),
        out_type=jax.ShapeDtypeStruct((dim, batch), jnp.float32),
        scratch_types=[
            pltpu.VMEM((b_per_w,), jnp.int32),
            pltpu.VMEM((b_per_w,), jnp.int32),
            pltpu.VMEM((b_per_w,), jnp.int32),
            [pltpu.VMEM((_G, _DQ, _LANES), jnp.float32) for _ in range(_NB)],
            pltpu.VMEM((dim, b_per_w), jnp.float32),
            [pltpu.SemaphoreType.DMA for _ in range(_NB)],
        ],
    )
    def k(idx_hbm, tt_hbm, out_hbm, idx_v, blks_v, lanes_v, banks, rows_v,
          sems):
        wid = lax.axis_index("s") * info.num_cores + lax.axis_index("c")
        base = wid * b_per_w
        pltpu.sync_copy(idx_hbm.at[pl.ds(base, b_per_w)], idx_v)
        d16 = lax.iota(jnp.int32, _G)

        def precompute(g):
            vec = idx_v[pl.ds(g * _G, _G)]
            blks_v[pl.ds(g * _G, _G)] = lax.shift_right_logical(vec, 7)
            lanes_v[pl.ds(g * _G, _G)] = lax.bitwise_and(vec, _LANES - 1)

        def fire(g, dq, b):
            bvec = blks_v[pl.ds(g * _G, _G)]
            for j in range(_G):
                col0 = pl.multiple_of(bvec[j] * _LANES, _LANES)
                pltpu.async_copy(
                    tt_hbm.at[pl.ds(dq * _DQ, _DQ), pl.ds(col0, _LANES)],
                    banks[b].at[j],
                    sems[b],
                )

        def drain(b):
            for _ in range(_G):
                pltpu.make_async_copy(
                    tt_hbm.at[pl.ds(0, _DQ), pl.ds(0, _LANES)],
                    banks[b].at[0],
                    sems[b],
                ).wait()

        def extract(g, dq, b):
            lvec = lanes_v[pl.ds(g * _G, _G)]
            for dl in range(_DQ):
                dvec = jnp.full((_G,), dl, jnp.int32)
                vals = plsc.load_gather(banks[b], [d16, dvec, lvec])
                rows_v[dq * _DQ + dl, pl.ds(g * _G, _G)] = vals

        for g in range(2):
            precompute(g)
        for u in range(_NB):
            fire(u // nq, u % nq, u % _NB)
        for g in range(2, n_groups):
            precompute(g)

        def body(v, _):
            for r in range(_UPB):
                b = r % _NB
                gq = 3 * v + r // nq
                drain(b)
                extract(gq, r % nq, b)
                uf = r + _NB
                fire(3 * v + uf // nq, uf % nq, b)
            return 0

        lax.fori_loop(0, n_bodies, body, 0)
        u0 = n_bodies * _UPB
        for u in range(u0, n_units):
            b = u % _NB
            drain(b)
            extract(u // nq, u % nq, b)
            if u + _NB < n_units:
                fire((u + _NB) // nq, (u + _NB) % nq, b)
        pltpu.sync_copy(rows_v, out_hbm.at[:, pl.ds(base, b_per_w)])

    return k


def kernel(x, table):
    batch = x.shape[0]
    num_rows, dim = table.shape
    outt = _make_gather(num_rows, dim, batch)(x, table.T)
    return outt.T


# speedup vs baseline: 21.6239x; 1.0238x over previous
import functools

import jax
import jax.numpy as jnp
from jax import lax
from jax.experimental import pallas as pl
from jax.experimental.pallas import tpu as pltpu
from jax.experimental.pallas import tpu_sc as plsc

_G = 16
_LANES = 128
_DQ = 8
_NB = 6
_UPB = 12


@functools.lru_cache(maxsize=None)
def _make_gather(num_rows, dim, batch):
    info = plsc.get_sparse_core_info()
    nw = info.num_cores * info.num_subcores
    b_per_w = batch // nw
    n_groups = b_per_w // _G
    nq = dim // _DQ
    n_units = n_groups * nq
    n_bodies = (n_units - _NB - (_UPB - _NB)) // _UPB
    mesh = plsc.VectorSubcoreMesh(core_axis_name="c", subcore_axis_name="s")

    @functools.partial(
        pl.kernel,
        mesh=mesh,
        compiler_params=pltpu.CompilerParams(needs_layout_passes=False),
        out_type=jax.ShapeDtypeStruct((dim, batch), jnp.float32),
        scratch_types=[
            pltpu.VMEM((b_per_w,), jnp.int32),
            pltpu.VMEM((b_per_w,), jnp.int32),
            pltpu.VMEM((b_per_w,), jnp.int32),
            [pltpu.VMEM((_G, _DQ, _LANES), jnp.float32) for _ in range(_NB)],
            pltpu.VMEM((dim, b_per_w), jnp.float32),
            [pltpu.SemaphoreType.DMA for _ in range(_NB)],
        ],
    )
    def k(idx_hbm, tt_hbm, out_hbm, idx_v, blks_v, lanes_v, banks, rows_v,
          sems):
        wid = lax.axis_index("s") * info.num_cores + lax.axis_index("c")
        base = wid * b_per_w
        pltpu.sync_copy(idx_hbm.at[pl.ds(base, b_per_w)], idx_v)
        d16 = lax.iota(jnp.int32, _G)

        def precompute(g):
            vec = idx_v[pl.ds(g * _G, _G)]
            blks_v[pl.ds(g * _G, _G)] = lax.shift_right_logical(vec, 7)
            lanes_v[pl.ds(g * _G, _G)] = lax.bitwise_and(vec, _LANES - 1)

        def fire(g, dq, b):
            bvec = blks_v[pl.ds(g * _G, _G)]
            for j in range(_G):
                col0 = pl.multiple_of(bvec[j] * _LANES, _LANES)
                pltpu.async_copy(
                    tt_hbm.at[pl.ds(dq * _DQ, _DQ), pl.ds(col0, _LANES)],
                    banks[b].at[j],
                    sems[b],
                )

        def drain(b):
            for _ in range(_G * _DQ // dim):
                pltpu.make_async_copy(
                    tt_hbm.at[pl.ds(0, dim), pl.ds(0, _LANES)],
                    rows_v.at[:, pl.ds(0, _LANES)],
                    sems[b],
                ).wait()

        def extract(g, dq, b):
            lvec = lanes_v[pl.ds(g * _G, _G)]
            for dl in range(_DQ):
                dvec = jnp.full((_G,), dl, jnp.int32)
                vals = plsc.load_gather(banks[b], [d16, dvec, lvec])
                rows_v[dq * _DQ + dl, pl.ds(g * _G, _G)] = vals

        for g in range(2):
            precompute(g)
        for u in range(_NB):
            fire(u // nq, u % nq, u % _NB)
        for g in range(2, n_groups):
            precompute(g)

        def body(v, _):
            for r in range(_UPB):
                b = r % _NB
                gq = 3 * v + r // nq
                drain(b)
                extract(gq, r % nq, b)
                uf = r + _NB
                fire(3 * v + uf // nq, uf % nq, b)
            return 0

        lax.fori_loop(0, n_bodies, body, 0)
        u0 = n_bodies * _UPB
        for u in range(u0, n_units):
            b = u % _NB
            drain(b)
            extract(u // nq, u % nq, b)
            if u + _NB < n_units:
                fire((u + _NB) // nq, (u + _NB) % nq, b)
        pltpu.sync_copy(rows_v, out_hbm.at[:, pl.ds(base, b_per_w)])

    return k


def kernel(x, table):
    batch = x.shape[0]
    num_rows, dim = table.shape
    outt = _make_gather(num_rows, dim, batch)(x, table.T)
    return outt.T


# speedup vs baseline: 21.6996x; 1.0035x over previous
import functools

import jax
import jax.numpy as jnp
from jax import lax
from jax.experimental import pallas as pl
from jax.experimental.pallas import tpu as pltpu
from jax.experimental.pallas import tpu_sc as plsc

_G = 16
_LANES = 128
_DQ = 8
_NB = 6
_UPB = 12


@functools.lru_cache(maxsize=None)
def _make_gather(num_rows, dim, batch):
    info = plsc.get_sparse_core_info()
    nw = info.num_cores * info.num_subcores
    b_per_w = batch // nw
    n_groups = b_per_w // _G
    nq = dim // _DQ
    n_units = n_groups * nq
    n_bodies = (n_units - _NB - (_UPB - _NB)) // _UPB
    mesh = plsc.VectorSubcoreMesh(core_axis_name="c", subcore_axis_name="s")

    @functools.partial(
        pl.kernel,
        mesh=mesh,
        compiler_params=pltpu.CompilerParams(needs_layout_passes=False),
        out_type=jax.ShapeDtypeStruct((dim, batch), jnp.float32),
        scratch_types=[
            pltpu.VMEM((b_per_w,), jnp.int32),
            pltpu.VMEM((b_per_w,), jnp.int32),
            pltpu.VMEM((b_per_w,), jnp.int32),
            [pltpu.VMEM((_G, _DQ, _LANES), jnp.float32) for _ in range(_NB)],
            pltpu.VMEM((dim, b_per_w), jnp.float32),
            [pltpu.SemaphoreType.DMA for _ in range(_NB)],
        ],
    )
    def k(idx_hbm, tt_hbm, out_hbm, idx_v, blks_v, lanes_v, banks, rows_v,
          sems):
        wid = lax.axis_index("s") * info.num_cores + lax.axis_index("c")
        base = wid * b_per_w
        pltpu.sync_copy(idx_hbm.at[pl.ds(base, b_per_w)], idx_v)
        d16 = lax.iota(jnp.int32, _G)

        def precompute(g):
            vec = idx_v[pl.ds(g * _G, _G)]
            blks_v[pl.ds(g * _G, _G)] = lax.shift_right_logical(vec, 7)
            lanes_v[pl.ds(g * _G, _G)] = lax.bitwise_and(vec, _LANES - 1)

        def fire(g, dq, b):
            bvec = blks_v[pl.ds(g * _G, _G)]
            for j in range(_G):
                col0 = pl.multiple_of(bvec[j] * _LANES, _LANES)
                pltpu.async_copy(
                    tt_hbm.at[pl.ds(dq * _DQ, _DQ), pl.ds(col0, _LANES)],
                    banks[b].at[j],
                    sems[b],
                )

        def drain(b):
            pltpu.make_async_copy(
                tt_hbm.at[pl.ds(0, dim), pl.ds(0, _G * _DQ * _LANES // dim)],
                rows_v,
                sems[b],
            ).wait()

        def extract(g, dq, b):
            lvec = lanes_v[pl.ds(g * _G, _G)]
            for dl in range(_DQ):
                dvec = jnp.full((_G,), dl, jnp.int32)
                vals = plsc.load_gather(banks[b], [d16, dvec, lvec])
                rows_v[dq * _DQ + dl, pl.ds(g * _G, _G)] = vals

        for g in range(2):
            precompute(g)
        for u in range(_NB):
            fire(u // nq, u % nq, u % _NB)
        for g in range(2, n_groups):
            precompute(g)

        def body(v, _):
            for r in range(_UPB):
                b = r % _NB
                gq = 3 * v + r // nq
                drain(b)
                extract(gq, r % nq, b)
                uf = r + _NB
                fire(3 * v + uf // nq, uf % nq, b)
            return 0

        lax.fori_loop(0, n_bodies, body, 0)
        u0 = n_bodies * _UPB
        for u in range(u0, n_units):
            b = u % _NB
            drain(b)
            extract(u // nq, u % nq, b)
            if u + _NB < n_units:
                fire((u + _NB) // nq, (u + _NB) % nq, b)
        pltpu.sync_copy(rows_v, out_hbm.at[:, pl.ds(base, b_per_w)])

    return k


def kernel(x, table):
    batch = x.shape[0]
    num_rows, dim = table.shape
    outt = _make_gather(num_rows, dim, batch)(x, table.T)
    return outt.T
